# Initial kernel scaffold; baseline (speedup 1.0000x reference)
import jax
import jax.numpy as jnp
from jax.experimental import pallas as pl


def kernel(h_a, x_a, e_a_idx, e_a_type, e_a_attr, h_f, x_f, e_f_idx, e_f_attr, m_mat, bm_mat, params):
    raise NotImplementedError("write your pallas kernel here")


# speedup vs baseline: 1.7380x; 1.7380x over previous
import functools

import jax
import jax.numpy as jnp
from jax import lax
from jax.experimental import pallas as pl
from jax.experimental.pallas import tpu as pltpu
from jax.experimental.pallas import tpu_sc as plsc

F32 = jnp.float32
NC, NS = 2, 16
NW = NC * NS
COORDS_RANGE = 15.0
INV_NORM = 0.01


def _cdiv(a, b):
    return (a + b - 1) // b


def _pad_rows(x, n):
    if x.shape[0] == n:
        return x
    pad = [(0, n - x.shape[0])] + [(0, 0)] * (x.ndim - 1)
    return jnp.pad(x, pad)


def _pad_len(e):
    for c in (128, 64, 32, 16, 8):
        ep = _cdiv(e, NW * c) * NW * c
        if ep - e <= max(e // 16, NW * 8):
            return ep, c
    return _cdiv(e, NW * 8) * NW * 8, 8


def _chunk_of(e):
    b = e // NW
    return max(c for c in (8, 16, 32, 64, 128) if b % c == 0)


def _sc_gathers(specs):
    plans = []
    flat_in = []
    out_type = []
    scratch = []
    for tables, idxs in specs:
        e = idxs[0].shape[0]
        d = tables[0].shape[1]
        b = e // NW
        c = _chunk_of(e)
        plans.append((e, d, b, c, b // c, len(tables)))
        flat_in += list(tables) + list(idxs)
        out_type.append(jax.ShapeDtypeStruct((e, d), F32))
        scratch.append([pltpu.VMEM((c,), jnp.int32) for _ in tables]
                       + [pltpu.VMEM((c, d), F32), pltpu.SemaphoreType.DMA])
    n_in = len(flat_in)
    mesh = plsc.VectorSubcoreMesh(core_axis_name="c", subcore_axis_name="s")
    flat_scratch = [s for group in scratch for s in group]

    @functools.partial(
        pl.kernel, out_type=tuple(out_type), mesh=mesh,
        scratch_types=flat_scratch,
        compiler_params=pltpu.CompilerParams(use_tc_tiling_on_sc=False))
    def run(*refs):
        wid = lax.axis_index("s") * NC + lax.axis_index("c")
        ipos = 0
        spos = n_in + len(plans)
        for k, (e, d, b, c, nch, ntab) in enumerate(plans):
            t_refs = refs[ipos:ipos + ntab]
            i_refs = refs[ipos + ntab:ipos + 2 * ntab]
            o_ref = refs[n_in + k]
            idx_vs = refs[spos:spos + ntab]
            buf_v = refs[spos + ntab]
            sem = refs[spos + ntab + 1]
            ipos += 2 * ntab
            spos += ntab + 2
            base = wid * b

            def step(j, _, t_refs=t_refs, i_refs=i_refs, o_ref=o_ref,
                     idx_vs=idx_vs, buf_v=buf_v, sem=sem, base=base, c=c,
                     ntab=ntab):
                off = pl.multiple_of(base + j * c, 8)
                for q in range(ntab):
                    pltpu.sync_copy(i_refs[q].at[pl.ds(off, c)], idx_vs[q])
                pltpu.async_copy(t_refs[0].at[idx_vs[0]], buf_v, sem).wait()
                if ntab == 2:
                    pltpu.async_copy(t_refs[1].at[idx_vs[1]], buf_v, sem,
                                     add=True).wait()
                pltpu.sync_copy(buf_v, o_ref.at[pl.ds(off, c)])
                return 0

            lax.fori_loop(0, nch, step, 0)

    outs = run(*flat_in)
    return outs if isinstance(outs, (tuple, list)) else (outs,)


def _sc_scatter_add(data, idx, n_rows):
    e, d = data.shape
    b = e // NW
    c = _chunk_of(e)
    nch = b // c
    np_rows = _cdiv(n_rows, NS * 8) * NS * 8
    rps = np_rows // NS
    zeros = jnp.zeros((rps, d), F32)
    mesh = plsc.VectorSubcoreMesh(core_axis_name="c", subcore_axis_name="s")

    @functools.partial(
        pl.kernel,
        out_type=jax.ShapeDtypeStruct((NC, np_rows, d), F32),
        mesh=mesh,
        scratch_types=[pltpu.VMEM((c,), jnp.int32), pltpu.VMEM((c, d), F32),
                       pltpu.VMEM_SHARED((np_rows, d), F32),
                       pltpu.SemaphoreType.DMA],
        compiler_params=pltpu.CompilerParams(use_tc_tiling_on_sc=False))
    def run(data_hbm, idx_hbm, zero_hbm, out_hbm, idx_v, buf_v, acc, sem):
        ci = lax.axis_index("c")
        si = lax.axis_index("s")
        wid = si * NC + ci
        row0 = pl.multiple_of(si * rps, 8)
        pltpu.sync_copy(zero_hbm, acc.at[pl.ds(row0, rps)])
        plsc.subcore_barrier()
        base = wid * b

        def step(j, _):
            off = pl.multiple_of(base + j * c, 8)
            pltpu.sync_copy(idx_hbm.at[pl.ds(off, c)], idx_v)
            pltpu.sync_copy(data_hbm.at[pl.ds(off, c)], buf_v)
            pltpu.sync_copy(buf_v, acc.at[idx_v], add=True)
            return 0

        lax.fori_loop(0, nch, step, 0)
        plsc.subcore_barrier()
        pltpu.sync_copy(acc.at[pl.ds(row0, rps)],
                        out_hbm.at[ci, pl.ds(row0, rps)])

    return run(data, idx, zeros)


def _assign_from_bm(bm):
    na, nf = bm.shape
    blk = 256
    grid = (_cdiv(na, blk),)

    def body(bm_ref, out_ref):
        col = lax.broadcasted_iota(jnp.int32, (blk, nf), 1).astype(F32)
        out_ref[...] = jnp.sum(bm_ref[...] * col, axis=1,
                               keepdims=True).astype(jnp.int32)

    out = pl.pallas_call(
        body, grid=grid,
        in_specs=[pl.BlockSpec((blk, nf), lambda i: (i, 0))],
        out_specs=pl.BlockSpec((blk, 1), lambda i: (i, 0)),
        out_shape=jax.ShapeDtypeStruct((na, 1), jnp.int32))(bm)
    return out[:, 0]


def _proj2(h, wst, wtt, b0):
    n, hd = h.shape
    blk = 512
    grid = (_cdiv(n, blk),)

    def body(h_ref, ws_ref, wt_ref, b_ref, p_ref, q_ref):
        hh = h_ref[...]
        p_ref[...] = jnp.dot(hh, ws_ref[...],
                             preferred_element_type=F32) + b_ref[...]
        q_ref[...] = jnp.dot(hh, wt_ref[...], preferred_element_type=F32)

    full = pl.BlockSpec((hd, hd), lambda i: (0, 0))
    return pl.pallas_call(
        body, grid=grid,
        in_specs=[pl.BlockSpec((blk, hd), lambda i: (i, 0)), full, full,
                  pl.BlockSpec((1, hd), lambda i: (0, 0))],
        out_specs=[pl.BlockSpec((blk, hd), lambda i: (i, 0))] * 2,
        out_shape=[jax.ShapeDtypeStruct((n, hd), F32)] * 2)(h, wst, wtt, b0)


def _edge_mlp(g12, et2, attr2, dxy, tbl, wa, w1t, b1, watt, batt, n_real,
              trans=False):
    ep, hd = g12.shape
    blk = 1024
    grid = (ep // blk,)
    has_type = tbl is not None
    nt = tbl.shape[0] if has_type else 0
    d_out = 16 if trans else hd

    def body(*refs):
        if has_type:
            (g_ref, et_ref, a_ref, d_ref, tbl_ref, wa_ref, w1_ref, b1_ref,
             watt_ref, batt_ref, o_ref) = refs
        else:
            (g_ref, a_ref, d_ref, wa_ref, w1_ref, b1_ref, watt_ref,
             batt_ref, o_ref) = refs
        i = pl.program_id(0)
        dd = d_ref[...]
        radial = jnp.sum(dd * dd, axis=1, keepdims=True)
        pre = g12_val = g_ref[...]
        if has_type:
            oh = (et_ref[...] == lax.broadcasted_iota(
                jnp.int32, (blk, nt), 1)).astype(F32)
            pre = pre + jnp.dot(oh, tbl_ref[...], preferred_element_type=F32)
        pre = pre + radial * wa_ref[0:1, :] + a_ref[...] * wa_ref[1:2, :]
        t = jax.nn.silu(pre)
        mij = jax.nn.silu(jnp.dot(t, w1_ref[...],
                                  preferred_element_type=F32) + b1_ref[...])
        rowid = i * blk + lax.broadcasted_iota(jnp.int32, (blk, 1), 0)
        mask = rowid < n_real
        if trans:
            s = jnp.dot(mij, watt_ref[...], preferred_element_type=F32)
            s = s + batt_ref[...]
            cd = dd / (jnp.sqrt(radial + 1e-8) + 1.0)
            out = cd * (jnp.tanh(s) * COORDS_RANGE)
        else:
            att = jax.nn.sigmoid(
                jnp.dot(mij, watt_ref[...], preferred_element_type=F32)
                + batt_ref[...])
            out = mij * att
        o_ref[...] = jnp.where(mask, out, 0.0)

    eblk = pl.BlockSpec((blk, hd), lambda i: (i, 0))
    one = pl.BlockSpec((blk, 1), lambda i: (i, 0))
    full = pl.BlockSpec((hd, hd), lambda i: (0, 0))
    in_specs = [eblk]
    args = [g12]
    if has_type:
        in_specs.append(one)
        args.append(et2)
    in_specs += [one, pl.BlockSpec((blk, 16), lambda i: (i, 0))]
    args += [attr2, dxy]
    if has_type:
        in_specs.append(pl.BlockSpec((nt, hd), lambda i: (0, 0)))
        args.append(tbl)
    in_specs += [pl.BlockSpec((2, hd), lambda i: (0, 0)), full,
                 pl.BlockSpec((1, hd), lambda i: (0, 0)),
                 pl.BlockSpec((hd, 1), lambda i: (0, 0)),
                 pl.BlockSpec((1, 1), lambda i: (0, 0))]
    args += [wa, w1t, b1, watt, batt]
    return pl.pallas_call(
        body, grid=grid, in_specs=in_specs,
        out_specs=pl.BlockSpec((blk, d_out), lambda i: (i, 0)),
        out_shape=jax.ShapeDtypeStruct((ep, d_out), F32))(*args)


def _node_mlp(h, aux, agg0, agg1, n0at, n0bt, n0ct, b0, n1t, b1):
    n, hd = h.shape
    blk = 512
    grid = (_cdiv(n, blk),)
    n_aux = len(aux)

    def body(*refs):
        h_ref = refs[0]
        aux_refs = refs[1:1 + n_aux]
        a0_ref, a1_ref, wa_ref, wb_ref, wc_ref, b0_ref, w1_ref, b1_ref, \
            o_ref = refs[1 + n_aux:]
        hh = h_ref[...]
        av = aux_refs[0][...]
        if n_aux == 2:
            av = av + aux_refs[1][...]
        agg = (a0_ref[...] + a1_ref[...]) * INV_NORM
        pre = (jnp.dot(hh, wa_ref[...], preferred_element_type=F32)
               + jnp.dot(av, wb_ref[...], preferred_element_type=F32)
               + jnp.dot(agg, wc_ref[...], preferred_element_type=F32)
               + b0_ref[...])
        t = jax.nn.silu(pre)
        o_ref[...] = hh + jnp.dot(t, w1_ref[...],
                                  preferred_element_type=F32) + b1_ref[...]

    rblk = pl.BlockSpec((blk, hd), lambda i: (i, 0))
    full = pl.BlockSpec((hd, hd), lambda i: (0, 0))
    bias = pl.BlockSpec((1, hd), lambda i: (0, 0))
    in_specs = [rblk] * (1 + n_aux) + [rblk, rblk, full, full, full, bias,
                                       full, bias]
    return pl.pallas_call(
        body, grid=grid, in_specs=in_specs, out_specs=rblk,
        out_shape=jax.ShapeDtypeStruct((n, hd), F32))(
            h, *aux, agg0, agg1, n0at, n0bt, n0ct, b0, n1t, b1)


def _final_coords(xa16, h_a, hfa, xfa16, at0, at1, f0at, f0bt, wfr, b0, f1t,
                  b1, f2w):
    n, hd = h_a.shape
    blk = 512
    grid = (_cdiv(n, blk),)

    def body(x_ref, h_ref, hf_ref, xf_ref, a0_ref, a1_ref, wa_ref, wb_ref,
             wr_ref, b0_ref, w1_ref, b1_ref, w2_ref, o_ref):
        x = x_ref[...]
        cdf = x - xf_ref[...]
        radial = jnp.sum(cdf * cdf, axis=1, keepdims=True)
        cdfn = cdf / (jnp.sqrt(radial + 1e-8) + 1.0)
        pre = (jnp.dot(h_ref[...], wa_ref[...], preferred_element_type=F32)
               + jnp.dot(hf_ref[...], wb_ref[...], preferred_element_type=F32)
               + radial * wr_ref[...] + b0_ref[...])
        t = jax.nn.silu(pre)
        u = jax.nn.silu(jnp.dot(t, w1_ref[...],
                                preferred_element_type=F32) + b1_ref[...])
        s = jnp.dot(u, w2_ref[...], preferred_element_type=F32)
        trans = cdfn * (jnp.tanh(s) * COORDS_RANGE)
        agg = (a0_ref[...] + a1_ref[...]) * INV_NORM
        o_ref[...] = x + agg + trans

    c16 = pl.BlockSpec((blk, 16), lambda i: (i, 0))
    rblk = pl.BlockSpec((blk, hd), lambda i: (i, 0))
    full = pl.BlockSpec((hd, hd), lambda i: (0, 0))
    bias = pl.BlockSpec((1, hd), lambda i: (0, 0))
    return pl.pallas_call(
        body, grid=grid,
        in_specs=[c16, rblk, rblk, c16, c16, c16, full, full, bias, bias,
                  full, bias, pl.BlockSpec((hd, 1), lambda i: (0, 0))],
        out_specs=c16,
        out_shape=jax.ShapeDtypeStruct((n, 16), F32))(
            xa16, h_a, hfa, xfa16, at0, at1, f0at, f0bt, wfr, b0, f1t, b1,
            f2w)


def _gcl_weights(p, hd, with_type):
    w0 = p['edge_mlp0']['w']
    ws = w0[:, :hd].T
    wt = w0[:, hd:2 * hd].T
    wa = w0[:, 2 * hd:2 * hd + 2].T
    b0 = p['edge_mlp0']['b'][None, :]
    tbl = None
    if with_type:
        wb = w0[:, 2 * hd + 2:]
        tbl = jnp.dot(p['bond_emb'], wb.T)
    w1t = p['edge_mlp1']['w'].T
    b1 = p['edge_mlp1']['b'][None, :]
    watt = p['att_mlp']['w'].T
    batt = p['att_mlp']['b'].reshape(1, 1)
    n0 = p['node_mlp0']['w']
    n0at = n0[:, :hd].T
    n0bt = n0[:, hd:2 * hd].T
    n0ct = n0[:, 2 * hd:].T
    nb0 = p['node_mlp0']['b'][None, :]
    n1t = p['node_mlp1']['w'].T
    nb1 = p['node_mlp1']['b'][None, :]
    return dict(ws=ws, wt=wt, wa=wa, b0=b0, tbl=tbl, w1t=w1t, b1=b1,
                watt=watt, batt=batt, n0at=n0at, n0bt=n0bt, n0ct=n0ct,
                nb0=nb0, n1t=n1t, nb1=nb1)


def kernel(h_a, x_a, e_a_idx, e_a_type, e_a_attr, h_f, x_f, e_f_idx,
           e_f_attr, m_mat, bm_mat, params):
    na, hd = h_a.shape
    nf = h_f.shape[0]
    ea = e_a_idx.shape[1]
    ef = e_f_idx.shape[1]
    nl = len(params['f_inv'])

    eap, _ = _pad_len(ea)
    efp, _ = _pad_len(ef)
    nap, _ = _pad_len(na)

    def pad1(v, n):
        return _pad_rows(v.astype(jnp.int32), n)

    row_a = pad1(e_a_idx[0], eap)
    col_a = pad1(e_a_idx[1], eap)
    row_f = pad1(e_f_idx[0], efp)
    col_f = pad1(e_f_idx[1], efp)
    et2 = _pad_rows(e_a_type.astype(jnp.int32)[:, None], eap)
    attr_a2 = _pad_rows(e_a_attr, eap)
    attr_f2 = _pad_rows(e_f_attr, efp)

    assign = _assign_from_bm(bm_mat)
    assign_p = pad1(assign, nap)

    xa16 = jnp.pad(x_a, ((0, 0), (0, 13)))
    xf16 = jnp.pad(x_f, ((0, 0), (0, 13)))

    dxy_a, dxy_f, xfa16 = _sc_gathers([
        ((xa16, -xa16), (row_a, col_a)),
        ((xf16, -xf16), (row_f, col_f)),
        ((xf16,), (assign_p,)),
    ])

    for i in range(nl):
        wf = _gcl_weights(params['f_inv'][i], hd, with_type=False)
        wa_ = _gcl_weights(params['a_inv'][i], hd, with_type=True)

        hs = _sc_scatter_add(_pad_rows(h_a, nap), assign_p, nf)

        pf, qf = _proj2(h_f, wf['ws'], wf['wt'], wf['b0'])
        (gf12,) = _sc_gathers([((pf, qf), (row_f, col_f))])
        eff = _edge_mlp(gf12, None, attr_f2, dxy_f, None, wf['wa'],
                        wf['w1t'], wf['b1'], wf['watt'], wf['batt'], ef)
        af = _sc_scatter_add(eff, row_f, nf)
        h_f = _node_mlp(h_f, (hs[0], hs[1]), af[0], af[1], wf['n0at'],
                        wf['n0bt'], wf['n0ct'], wf['nb0'], wf['n1t'],
                        wf['nb1'])

        pa, qa = _proj2(h_a, wa_['ws'], wa_['wt'], wa_['b0'])
        ga12, hfa = _sc_gathers([((pa, qa), (row_a, col_a)),
                                 ((h_f,), (assign_p,))])
        eaf = _edge_mlp(ga12, et2, attr_a2, dxy_a, wa_['tbl'], wa_['wa'],
                        wa_['w1t'], wa_['b1'], wa_['watt'], wa_['batt'], ea)
        aa = _sc_scatter_add(eaf, row_a, na)
        h_a = _node_mlp(h_a, (hfa,), aa[0], aa[1], wa_['n0at'], wa_['n0bt'],
                        wa_['n0ct'], wa_['nb0'], wa_['n1t'], wa_['nb1'])

    pe = params['a_eq']
    w0 = pe['atom0']['w']
    wse = w0[:, :hd].T
    wte = w0[:, hd:2 * hd].T
    wae = w0[:, 2 * hd:2 * hd + 2].T
    tbl_e = jnp.dot(pe['bond_emb'], w0[:, 2 * hd + 2:].T)
    b0e = pe['atom0']['b'][None, :]
    a1t = pe['atom1']['w'].T
    ab1 = pe['atom1']['b'][None, :]
    a2w = pe['atom2_w'].T
    zb = jnp.zeros((1, 1), F32)

    p_e, q_e = _proj2(h_a, wse, wte, b0e)
    ge12, hfa_fin = _sc_gathers([((p_e, q_e), (row_a, col_a)),
                                 ((h_f,), (assign_p,))])
    tr = _edge_mlp(ge12, et2, attr_a2, dxy_a, tbl_e, wae, a1t, ab1, a2w, zb,
                   ea, trans=True)
    at = _sc_scatter_add(tr, row_a, na)

    f0 = pe['frag0']['w']
    f0at = f0[:, :hd].T
    f0bt = f0[:, hd:2 * hd].T
    wfr = (f0[:, 2 * hd] + f0[:, 2 * hd + 1])[None, :]
    fb0 = pe['frag0']['b'][None, :]
    f1t = pe['frag1']['w'].T
    fb1 = pe['frag1']['b'][None, :]
    f2w = pe['frag2_w'].T

    xout16 = _final_coords(xa16, h_a, hfa_fin, xfa16, at[0], at[1], f0at,
                           f0bt, wfr, fb0, f1t, fb1, f2w)
    return h_a, xout16[:, :3]


# speedup vs baseline: 1.9132x; 1.1008x over previous
import functools

import jax
import jax.numpy as jnp
from jax import lax
from jax.experimental import pallas as pl
from jax.experimental.pallas import tpu as pltpu
from jax.experimental.pallas import tpu_sc as plsc

F32 = jnp.float32
NC, NS = 2, 16
NW = NC * NS
COORDS_RANGE = 15.0
INV_NORM = 0.01


def _cdiv(a, b):
    return (a + b - 1) // b


def _pad_rows(x, n):
    if x.shape[0] == n:
        return x
    pad = [(0, n - x.shape[0])] + [(0, 0)] * (x.ndim - 1)
    return jnp.pad(x, pad)


def _pad_len(e):
    for c in (128, 64, 32, 16, 8):
        ep = _cdiv(e, NW * c) * NW * c
        if ep - e <= max(e // 16, NW * 8):
            return ep, c
    return _cdiv(e, NW * 8) * NW * 8, 8


def _chunk_of(e):
    b = e // NW
    return max(c for c in (8, 16, 32, 64, 128) if b % c == 0)


_SPMEM_BUDGET = 6_800_000


def _group_k(n, d, c, reserved=0):
    cap = 8 if d <= 32 else 4
    per_k = NS * c * d * 4
    fit = max(1, (_SPMEM_BUDGET - reserved) // per_k)
    return max(1, min(cap, n, fit))


def _sc_gathers(specs):
    plans = []
    flat_in = []
    out_type = []
    scratch = []
    used = 0
    for tables, idxs in specs:
        e = idxs[0].shape[0]
        d = tables[0].shape[1]
        b = e // NW
        c = _chunk_of(e)
        nch = b // c
        k = _group_k(nch, d, c, reserved=used)
        used += NS * k * c * d * 4
        plans.append((e, d, b, c, nch, k, len(tables)))
        flat_in += list(tables) + [ix.reshape(e // c, c) for ix in idxs]
        out_type.append(jax.ShapeDtypeStruct((e, d), F32))
        scratch.append([pltpu.VMEM((k, c), jnp.int32) for _ in tables]
                       + [pltpu.VMEM((k * c, d), F32),
                          pltpu.SemaphoreType.DMA])
    n_in = len(flat_in)
    mesh = plsc.VectorSubcoreMesh(core_axis_name="c", subcore_axis_name="s")
    flat_scratch = [s for group in scratch for s in group]

    @functools.partial(
        pl.kernel, out_type=tuple(out_type), mesh=mesh,
        scratch_types=flat_scratch,
        compiler_params=pltpu.CompilerParams(use_tc_tiling_on_sc=False))
    def run(*refs):
        wid = lax.axis_index("s") * NC + lax.axis_index("c")
        ipos = 0
        spos = n_in + len(plans)
        for kk, (e, d, b, c, nch, k, ntab) in enumerate(plans):
            t_refs = refs[ipos:ipos + ntab]
            i_refs = refs[ipos + ntab:ipos + 2 * ntab]
            o_ref = refs[n_in + kk]
            idx_vs = refs[spos:spos + ntab]
            buf_v = refs[spos + ntab]
            sem = refs[spos + ntab + 1]
            ipos += 2 * ntab
            spos += ntab + 2

            def group(g, _, t_refs=t_refs, i_refs=i_refs, o_ref=o_ref,
                      idx_vs=idx_vs, buf_v=buf_v, sem=sem, wid=wid, b=b,
                      c=c, nch=nch, k=k, ntab=ntab, kc=None):
                row0 = wid * nch + g * k
                goff = pl.multiple_of(wid * b + g * (k * c), 8)
                for q in range(ntab):
                    pltpu.sync_copy(i_refs[q].at[pl.ds(row0, k)], idx_vs[q])
                waves = ([(0, False)] if ntab == 1
                         else [(0, False), (1, True)])
                for q, add in waves:
                    descs = [pltpu.async_copy(
                        t_refs[q].at[idx_vs[q].at[bq]],
                        buf_v.at[pl.ds(bq * c, c)], sem, add=add)
                        for bq in range(k)]
                    for dsc in descs:
                        dsc.wait()
                pltpu.sync_copy(buf_v, o_ref.at[pl.ds(goff, k * c)])
                return 0

            n_grp = nch // k
            rem = nch - n_grp * k
            lax.fori_loop(0, n_grp, group, 0)
            if rem:
                def tail(t_refs=t_refs, i_refs=i_refs, o_ref=o_ref,
                         idx_vs=idx_vs, buf_v=buf_v, sem=sem):
                    row0 = wid * nch + n_grp * k
                    goff = pl.multiple_of(wid * b + n_grp * (k * c), 8)
                    for q in range(ntab):
                        pltpu.sync_copy(i_refs[q].at[pl.ds(row0, rem)],
                                        idx_vs[q].at[pl.ds(0, rem)])
                    waves = ([(0, False)] if ntab == 1
                             else [(0, False), (1, True)])
                    for q, add in waves:
                        descs = [pltpu.async_copy(
                            t_refs[q].at[idx_vs[q].at[bq]],
                            buf_v.at[pl.ds(bq * c, c)], sem, add=add)
                            for bq in range(rem)]
                        for dsc in descs:
                            dsc.wait()
                    pltpu.sync_copy(buf_v.at[pl.ds(0, rem * c)],
                                    o_ref.at[pl.ds(goff, rem * c)])
                tail()

    outs = run(*flat_in)
    return outs if isinstance(outs, (tuple, list)) else (outs,)


def _sc_scatter_add(data, idx, n_rows):
    e, d = data.shape
    b = e // NW
    c = _chunk_of(e)
    nch = b // c
    np_rows = _cdiv(n_rows, NS * 8) * NS * 8
    rps = np_rows // NS
    zeros = jnp.zeros((rps, d), F32)
    mesh = plsc.VectorSubcoreMesh(core_axis_name="c", subcore_axis_name="s")

    k = _group_k(nch, d, c, reserved=np_rows * d * 4)
    n_grp = nch // k
    rem = nch - n_grp * k

    @functools.partial(
        pl.kernel,
        out_type=jax.ShapeDtypeStruct((NC, np_rows, d), F32),
        mesh=mesh,
        scratch_types=[pltpu.VMEM((k, c), jnp.int32),
                       pltpu.VMEM((k * c, d), F32),
                       pltpu.VMEM_SHARED((np_rows, d), F32),
                       pltpu.SemaphoreType.DMA, pltpu.SemaphoreType.DMA],
        compiler_params=pltpu.CompilerParams(use_tc_tiling_on_sc=False))
    def run(data_hbm, idx_hbm, zero_hbm, out_hbm, idx_v, buf_v, acc, sem,
            sem2):
        ci = lax.axis_index("c")
        si = lax.axis_index("s")
        wid = si * NC + ci
        row0 = pl.multiple_of(si * rps, 8)
        pltpu.sync_copy(zero_hbm, acc.at[pl.ds(row0, rps)])
        plsc.subcore_barrier()

        def emit(g_row0, goff, kk):
            pltpu.sync_copy(idx_hbm.at[pl.ds(g_row0, kk)],
                            idx_v.at[pl.ds(0, kk)])
            pltpu.async_copy(data_hbm.at[pl.ds(goff, kk * c)],
                             buf_v.at[pl.ds(0, kk * c)], sem).wait()
            descs = [pltpu.async_copy(buf_v.at[pl.ds(bq * c, c)],
                                      acc.at[idx_v.at[bq]], sem2, add=True)
                     for bq in range(kk)]
            for dsc in descs:
                dsc.wait()

        def group(g, _):
            g_row0 = wid * nch + g * k
            goff = pl.multiple_of(wid * b + g * (k * c), 8)
            emit(g_row0, goff, k)
            return 0

        lax.fori_loop(0, n_grp, group, 0)
        if rem:
            emit(wid * nch + n_grp * k,
                 pl.multiple_of(wid * b + n_grp * (k * c), 8), rem)
        plsc.subcore_barrier()
        pltpu.sync_copy(acc.at[pl.ds(row0, rps)],
                        out_hbm.at[ci, pl.ds(row0, rps)])

    return run(data, idx.reshape(e // c, c), zeros)


def _assign_from_bm(bm):
    na, nf = bm.shape
    blk = 256
    grid = (_cdiv(na, blk),)

    def body(bm_ref, out_ref):
        col = lax.broadcasted_iota(jnp.int32, (blk, nf), 1).astype(F32)
        out_ref[...] = jnp.sum(bm_ref[...] * col, axis=1,
                               keepdims=True).astype(jnp.int32)

    out = pl.pallas_call(
        body, grid=grid,
        in_specs=[pl.BlockSpec((blk, nf), lambda i: (i, 0))],
        out_specs=pl.BlockSpec((blk, 1), lambda i: (i, 0)),
        out_shape=jax.ShapeDtypeStruct((na, 1), jnp.int32))(bm)
    return out[:, 0]


def _proj2(h, wst, wtt, b0):
    n, hd = h.shape
    blk = 512
    grid = (_cdiv(n, blk),)

    def body(h_ref, ws_ref, wt_ref, b_ref, p_ref, q_ref):
        hh = h_ref[...]
        p_ref[...] = jnp.dot(hh, ws_ref[...],
                             preferred_element_type=F32) + b_ref[...]
        q_ref[...] = jnp.dot(hh, wt_ref[...], preferred_element_type=F32)

    full = pl.BlockSpec((hd, hd), lambda i: (0, 0))
    return pl.pallas_call(
        body, grid=grid,
        in_specs=[pl.BlockSpec((blk, hd), lambda i: (i, 0)), full, full,
                  pl.BlockSpec((1, hd), lambda i: (0, 0))],
        out_specs=[pl.BlockSpec((blk, hd), lambda i: (i, 0))] * 2,
        out_shape=[jax.ShapeDtypeStruct((n, hd), F32)] * 2)(h, wst, wtt, b0)


def _edge_mlp(g12, et2, attr2, dxy, tbl, wa, w1t, b1, watt, batt, n_real,
              trans=False):
    ep, hd = g12.shape
    blk = 1024
    grid = (ep // blk,)
    has_type = tbl is not None
    nt = tbl.shape[0] if has_type else 0
    d_out = 16 if trans else hd

    def body(*refs):
        if has_type:
            (g_ref, et_ref, a_ref, d_ref, tbl_ref, wa_ref, w1_ref, b1_ref,
             watt_ref, batt_ref, o_ref) = refs
        else:
            (g_ref, a_ref, d_ref, wa_ref, w1_ref, b1_ref, watt_ref,
             batt_ref, o_ref) = refs
        i = pl.program_id(0)
        dd = d_ref[...]
        radial = jnp.sum(dd * dd, axis=1, keepdims=True)
        pre = g12_val = g_ref[...]
        if has_type:
            oh = (et_ref[...] == lax.broadcasted_iota(
                jnp.int32, (blk, nt), 1)).astype(F32)
            pre = pre + jnp.dot(oh, tbl_ref[...], preferred_element_type=F32)
        pre = pre + radial * wa_ref[0:1, :] + a_ref[...] * wa_ref[1:2, :]
        t = jax.nn.silu(pre)
        mij = jax.nn.silu(jnp.dot(t, w1_ref[...],
                                  preferred_element_type=F32) + b1_ref[...])
        rowid = i * blk + lax.broadcasted_iota(jnp.int32, (blk, 1), 0)
        mask = rowid < n_real
        if trans:
            s = jnp.dot(mij, watt_ref[...], preferred_element_type=F32)
            s = s + batt_ref[...]
            cd = dd / (jnp.sqrt(radial + 1e-8) + 1.0)
            out = cd * (jnp.tanh(s) * COORDS_RANGE)
        else:
            att = jax.nn.sigmoid(
                jnp.dot(mij, watt_ref[...], preferred_element_type=F32)
                + batt_ref[...])
            out = mij * att
        o_ref[...] = jnp.where(mask, out, 0.0)

    eblk = pl.BlockSpec((blk, hd), lambda i: (i, 0))
    one = pl.BlockSpec((blk, 1), lambda i: (i, 0))
    full = pl.BlockSpec((hd, hd), lambda i: (0, 0))
    in_specs = [eblk]
    args = [g12]
    if has_type:
        in_specs.append(one)
        args.append(et2)
    in_specs += [one, pl.BlockSpec((blk, 16), lambda i: (i, 0))]
    args += [attr2, dxy]
    if has_type:
        in_specs.append(pl.BlockSpec((nt, hd), lambda i: (0, 0)))
        args.append(tbl)
    in_specs += [pl.BlockSpec((2, hd), lambda i: (0, 0)), full,
                 pl.BlockSpec((1, hd), lambda i: (0, 0)),
                 pl.BlockSpec((hd, 1), lambda i: (0, 0)),
                 pl.BlockSpec((1, 1), lambda i: (0, 0))]
    args += [wa, w1t, b1, watt, batt]
    return pl.pallas_call(
        body, grid=grid, in_specs=in_specs,
        out_specs=pl.BlockSpec((blk, d_out), lambda i: (i, 0)),
        out_shape=jax.ShapeDtypeStruct((ep, d_out), F32))(*args)


def _node_mlp(h, aux, agg0, agg1, n0at, n0bt, n0ct, b0, n1t, b1):
    n, hd = h.shape
    blk = 512
    grid = (_cdiv(n, blk),)
    n_aux = len(aux)

    def body(*refs):
        h_ref = refs[0]
        aux_refs = refs[1:1 + n_aux]
        a0_ref, a1_ref, wa_ref, wb_ref, wc_ref, b0_ref, w1_ref, b1_ref, \
            o_ref = refs[1 + n_aux:]
        hh = h_ref[...]
        av = aux_refs[0][...]
        if n_aux == 2:
            av = av + aux_refs[1][...]
        agg = (a0_ref[...] + a1_ref[...]) * INV_NORM
        pre = (jnp.dot(hh, wa_ref[...], preferred_element_type=F32)
               + jnp.dot(av, wb_ref[...], preferred_element_type=F32)
               + jnp.dot(agg, wc_ref[...], preferred_element_type=F32)
               + b0_ref[...])
        t = jax.nn.silu(pre)
        o_ref[...] = hh + jnp.dot(t, w1_ref[...],
                                  preferred_element_type=F32) + b1_ref[...]

    rblk = pl.BlockSpec((blk, hd), lambda i: (i, 0))
    full = pl.BlockSpec((hd, hd), lambda i: (0, 0))
    bias = pl.BlockSpec((1, hd), lambda i: (0, 0))
    in_specs = [rblk] * (1 + n_aux) + [rblk, rblk, full, full, full, bias,
                                       full, bias]
    return pl.pallas_call(
        body, grid=grid, in_specs=in_specs, out_specs=rblk,
        out_shape=jax.ShapeDtypeStruct((n, hd), F32))(
            h, *aux, agg0, agg1, n0at, n0bt, n0ct, b0, n1t, b1)


def _final_coords(xa16, h_a, hfa, xfa16, at0, at1, f0at, f0bt, wfr, b0, f1t,
                  b1, f2w):
    n, hd = h_a.shape
    blk = 512
    grid = (_cdiv(n, blk),)

    def body(x_ref, h_ref, hf_ref, xf_ref, a0_ref, a1_ref, wa_ref, wb_ref,
             wr_ref, b0_ref, w1_ref, b1_ref, w2_ref, o_ref):
        x = x_ref[...]
        cdf = x - xf_ref[...]
        radial = jnp.sum(cdf * cdf, axis=1, keepdims=True)
        cdfn = cdf / (jnp.sqrt(radial + 1e-8) + 1.0)
        pre = (jnp.dot(h_ref[...], wa_ref[...], preferred_element_type=F32)
               + jnp.dot(hf_ref[...], wb_ref[...], preferred_element_type=F32)
               + radial * wr_ref[...] + b0_ref[...])
        t = jax.nn.silu(pre)
        u = jax.nn.silu(jnp.dot(t, w1_ref[...],
                                preferred_element_type=F32) + b1_ref[...])
        s = jnp.dot(u, w2_ref[...], preferred_element_type=F32)
        trans = cdfn * (jnp.tanh(s) * COORDS_RANGE)
        agg = (a0_ref[...] + a1_ref[...]) * INV_NORM
        o_ref[...] = x + agg + trans

    c16 = pl.BlockSpec((blk, 16), lambda i: (i, 0))
    rblk = pl.BlockSpec((blk, hd), lambda i: (i, 0))
    full = pl.BlockSpec((hd, hd), lambda i: (0, 0))
    bias = pl.BlockSpec((1, hd), lambda i: (0, 0))
    return pl.pallas_call(
        body, grid=grid,
        in_specs=[c16, rblk, rblk, c16, c16, c16, full, full, bias, bias,
                  full, bias, pl.BlockSpec((hd, 1), lambda i: (0, 0))],
        out_specs=c16,
        out_shape=jax.ShapeDtypeStruct((n, 16), F32))(
            xa16, h_a, hfa, xfa16, at0, at1, f0at, f0bt, wfr, b0, f1t, b1,
            f2w)


def _gcl_weights(p, hd, with_type):
    w0 = p['edge_mlp0']['w']
    ws = w0[:, :hd].T
    wt = w0[:, hd:2 * hd].T
    wa = w0[:, 2 * hd:2 * hd + 2].T
    b0 = p['edge_mlp0']['b'][None, :]
    tbl = None
    if with_type:
        wb = w0[:, 2 * hd + 2:]
        tbl = jnp.dot(p['bond_emb'], wb.T)
    w1t = p['edge_mlp1']['w'].T
    b1 = p['edge_mlp1']['b'][None, :]
    watt = p['att_mlp']['w'].T
    batt = p['att_mlp']['b'].reshape(1, 1)
    n0 = p['node_mlp0']['w']
    n0at = n0[:, :hd].T
    n0bt = n0[:, hd:2 * hd].T
    n0ct = n0[:, 2 * hd:].T
    nb0 = p['node_mlp0']['b'][None, :]
    n1t = p['node_mlp1']['w'].T
    nb1 = p['node_mlp1']['b'][None, :]
    return dict(ws=ws, wt=wt, wa=wa, b0=b0, tbl=tbl, w1t=w1t, b1=b1,
                watt=watt, batt=batt, n0at=n0at, n0bt=n0bt, n0ct=n0ct,
                nb0=nb0, n1t=n1t, nb1=nb1)


def kernel(h_a, x_a, e_a_idx, e_a_type, e_a_attr, h_f, x_f, e_f_idx,
           e_f_attr, m_mat, bm_mat, params):
    na, hd = h_a.shape
    nf = h_f.shape[0]
    ea = e_a_idx.shape[1]
    ef = e_f_idx.shape[1]
    nl = len(params['f_inv'])

    eap, _ = _pad_len(ea)
    efp, _ = _pad_len(ef)
    nap, _ = _pad_len(na)

    def pad1(v, n):
        return _pad_rows(v.astype(jnp.int32), n)

    row_a = pad1(e_a_idx[0], eap)
    col_a = pad1(e_a_idx[1], eap)
    row_f = pad1(e_f_idx[0], efp)
    col_f = pad1(e_f_idx[1], efp)
    et2 = _pad_rows(e_a_type.astype(jnp.int32)[:, None], eap)
    attr_a2 = _pad_rows(e_a_attr, eap)
    attr_f2 = _pad_rows(e_f_attr, efp)

    assign = _assign_from_bm(bm_mat)
    assign_p = pad1(assign, nap)

    xa16 = jnp.pad(x_a, ((0, 0), (0, 13)))
    xf16 = jnp.pad(x_f, ((0, 0), (0, 13)))

    dxy_a, dxy_f, xfa16 = _sc_gathers([
        ((xa16, -xa16), (row_a, col_a)),
        ((xf16, -xf16), (row_f, col_f)),
        ((xf16,), (assign_p,)),
    ])

    for i in range(nl):
        wf = _gcl_weights(params['f_inv'][i], hd, with_type=False)
        wa_ = _gcl_weights(params['a_inv'][i], hd, with_type=True)

        hs = _sc_scatter_add(_pad_rows(h_a, nap), assign_p, nf)

        pf, qf = _proj2(h_f, wf['ws'], wf['wt'], wf['b0'])
        (gf12,) = _sc_gathers([((pf, qf), (row_f, col_f))])
        eff = _edge_mlp(gf12, None, attr_f2, dxy_f, None, wf['wa'],
                        wf['w1t'], wf['b1'], wf['watt'], wf['batt'], ef)
        af = _sc_scatter_add(eff, row_f, nf)
        h_f = _node_mlp(h_f, (hs[0], hs[1]), af[0], af[1], wf['n0at'],
                        wf['n0bt'], wf['n0ct'], wf['nb0'], wf['n1t'],
                        wf['nb1'])

        pa, qa = _proj2(h_a, wa_['ws'], wa_['wt'], wa_['b0'])
        ga12, hfa = _sc_gathers([((pa, qa), (row_a, col_a)),
                                 ((h_f,), (assign_p,))])
        eaf = _edge_mlp(ga12, et2, attr_a2, dxy_a, wa_['tbl'], wa_['wa'],
                        wa_['w1t'], wa_['b1'], wa_['watt'], wa_['batt'], ea)
        aa = _sc_scatter_add(eaf, row_a, na)
        h_a = _node_mlp(h_a, (hfa,), aa[0], aa[1], wa_['n0at'], wa_['n0bt'],
                        wa_['n0ct'], wa_['nb0'], wa_['n1t'], wa_['nb1'])

    pe = params['a_eq']
    w0 = pe['atom0']['w']
    wse = w0[:, :hd].T
    wte = w0[:, hd:2 * hd].T
    wae = w0[:, 2 * hd:2 * hd + 2].T
    tbl_e = jnp.dot(pe['bond_emb'], w0[:, 2 * hd + 2:].T)
    b0e = pe['atom0']['b'][None, :]
    a1t = pe['atom1']['w'].T
    ab1 = pe['atom1']['b'][None, :]
    a2w = pe['atom2_w'].T
    zb = jnp.zeros((1, 1), F32)

    p_e, q_e = _proj2(h_a, wse, wte, b0e)
    ge12, hfa_fin = _sc_gathers([((p_e, q_e), (row_a, col_a)),
                                 ((h_f,), (assign_p,))])
    tr = _edge_mlp(ge12, et2, attr_a2, dxy_a, tbl_e, wae, a1t, ab1, a2w, zb,
                   ea, trans=True)
    at = _sc_scatter_add(tr, row_a, na)

    f0 = pe['frag0']['w']
    f0at = f0[:, :hd].T
    f0bt = f0[:, hd:2 * hd].T
    wfr = (f0[:, 2 * hd] + f0[:, 2 * hd + 1])[None, :]
    fb0 = pe['frag0']['b'][None, :]
    f1t = pe['frag1']['w'].T
    fb1 = pe['frag1']['b'][None, :]
    f2w = pe['frag2_w'].T

    xout16 = _final_coords(xa16, h_a, hfa_fin, xfa16, at[0], at[1], f0at,
                           f0bt, wfr, fb0, f1t, fb1, f2w)
    return h_a, xout16[:, :3]


# speedup vs baseline: 2.0388x; 1.0657x over previous
import functools

import jax
import jax.numpy as jnp
from jax import lax
from jax.experimental import pallas as pl
from jax.experimental.pallas import tpu as pltpu
from jax.experimental.pallas import tpu_sc as plsc

F32 = jnp.float32
NC, NS = 2, 16
NW = NC * NS
COORDS_RANGE = 15.0
INV_NORM = 0.01


def _cdiv(a, b):
    return (a + b - 1) // b


def _pad_rows(x, n):
    if x.shape[0] == n:
        return x
    pad = [(0, n - x.shape[0])] + [(0, 0)] * (x.ndim - 1)
    return jnp.pad(x, pad)


def _pad_len(e):
    for c in (128, 64, 32, 16, 8):
        ep = _cdiv(e, NW * c) * NW * c
        if ep - e <= max(e // 16, NW * 8):
            return ep, c
    return _cdiv(e, NW * 8) * NW * 8, 8


def _chunk_of(e):
    b = e // NW
    return max(c for c in (8, 16, 32, 64, 128) if b % c == 0)


_SPMEM_BUDGET = 6_600_000


def _pick_chunk(b, d, budget, slots):
    cmax = budget // (NS * slots * (d * 4 + 4))
    best = 8
    for c in range(8, min(128, b) + 1, 8):
        if b % c == 0 and c <= cmax:
            best = c
    return best


def _sc_gathers(specs):
    plans = []
    flat_in = []
    out_type = []
    scratch = []
    budget = _SPMEM_BUDGET
    for tables, idxs in specs:
        e = idxs[0].shape[0]
        d = tables[0].shape[1]
        b = e // NW
        c = _pick_chunk(b, d, budget, slots=2)
        nch = b // c
        slots = 2 if nch >= 2 else 1
        budget -= NS * slots * c * (d * 4 + 4)
        plans.append((e, d, b, c, nch, slots, len(tables)))
        flat_in += list(tables) + list(idxs)
        out_type.append(jax.ShapeDtypeStruct((e, d), F32))
        group = []
        for _ in range(slots):
            group += [pltpu.VMEM((c,), jnp.int32) for _ in tables]
            group += [pltpu.VMEM((c, d), F32), pltpu.SemaphoreType.DMA,
                      pltpu.SemaphoreType.DMA]
        scratch.append(group)
    n_in = len(flat_in)
    mesh = plsc.VectorSubcoreMesh(core_axis_name="c", subcore_axis_name="s")
    flat_scratch = [s for group in scratch for s in group]

    @functools.partial(
        pl.kernel, out_type=tuple(out_type), mesh=mesh,
        scratch_types=flat_scratch,
        compiler_params=pltpu.CompilerParams(use_tc_tiling_on_sc=False))
    def run(*refs):
        wid = lax.axis_index("s") * NC + lax.axis_index("c")
        ipos = 0
        spos = n_in + len(plans)
        for kk, (e, d, b, c, nch, slots, ntab) in enumerate(plans):
            t_refs = refs[ipos:ipos + ntab]
            i_refs = refs[ipos + ntab:ipos + 2 * ntab]
            o_ref = refs[n_in + kk]
            per = ntab + 3
            slot_refs = [refs[spos + si * per:spos + (si + 1) * per]
                         for si in range(slots)]
            ipos += 2 * ntab
            spos += slots * per
            base = wid * b

            def do_pair(j0, t_refs=t_refs, i_refs=i_refs, o_ref=o_ref,
                        slot_refs=slot_refs, base=base, c=c, ntab=ntab,
                        slots=slots):
                descs = []
                for si in range(slots):
                    sr = slot_refs[si]
                    off = pl.multiple_of(base + (j0 + si) * c, 8)
                    for q in range(ntab):
                        pltpu.sync_copy(i_refs[q].at[pl.ds(off, c)], sr[q])
                    descs.append(pltpu.async_copy(t_refs[0].at[sr[0]],
                                                  sr[ntab], sr[ntab + 1]))
                if ntab == 2:
                    add_d = []
                    for si in range(slots):
                        sr = slot_refs[si]
                        descs[si].wait()
                        add_d.append(pltpu.async_copy(
                            t_refs[1].at[sr[1]], sr[ntab], sr[ntab + 1],
                            add=True))
                    descs = add_d
                wb = []
                for si in range(slots):
                    sr = slot_refs[si]
                    off = pl.multiple_of(base + (j0 + si) * c, 8)
                    descs[si].wait()
                    wb.append(pltpu.async_copy(
                        sr[ntab], o_ref.at[pl.ds(off, c)], sr[ntab + 2]))
                for dsc in wb:
                    dsc.wait()

            if slots == 2:
                lax.fori_loop(0, nch // 2,
                              lambda t, _: (do_pair(t * 2), 0)[1], 0)
                if nch % 2:
                    sr = slot_refs[0]
                    off = pl.multiple_of(base + (nch - 1) * c, 8)
                    for q in range(ntab):
                        pltpu.sync_copy(i_refs[q].at[pl.ds(off, c)], sr[q])
                    pltpu.async_copy(t_refs[0].at[sr[0]], sr[ntab],
                                     sr[ntab + 1]).wait()
                    if ntab == 2:
                        pltpu.async_copy(t_refs[1].at[sr[1]], sr[ntab],
                                         sr[ntab + 1], add=True).wait()
                    pltpu.sync_copy(sr[ntab], o_ref.at[pl.ds(off, c)])
            else:
                sr = slot_refs[0]
                off = pl.multiple_of(base, 8)
                for q in range(ntab):
                    pltpu.sync_copy(i_refs[q].at[pl.ds(off, c)], sr[q])
                pltpu.async_copy(t_refs[0].at[sr[0]], sr[ntab],
                                 sr[ntab + 1]).wait()
                if ntab == 2:
                    pltpu.async_copy(t_refs[1].at[sr[1]], sr[ntab],
                                     sr[ntab + 1], add=True).wait()
                pltpu.sync_copy(sr[ntab], o_ref.at[pl.ds(off, c)])

    outs = run(*flat_in)
    return outs if isinstance(outs, (tuple, list)) else (outs,)


def _sc_scatter_add(data, idx, n_rows):
    e, d = data.shape
    b = e // NW
    np_rows = _cdiv(n_rows, NS * 8) * NS * 8
    rps = np_rows // NS
    c = _pick_chunk(b, d, _SPMEM_BUDGET - np_rows * d * 4, slots=2)
    nch = b // c
    slots = 2 if nch >= 2 else 1
    zeros = jnp.zeros((rps, d), F32)
    mesh = plsc.VectorSubcoreMesh(core_axis_name="c", subcore_axis_name="s")

    scr = []
    for _ in range(slots):
        scr += [pltpu.VMEM((c,), jnp.int32), pltpu.VMEM((c, d), F32),
                pltpu.SemaphoreType.DMA, pltpu.SemaphoreType.DMA]
    scr.append(pltpu.VMEM_SHARED((np_rows, d), F32))

    @functools.partial(
        pl.kernel,
        out_type=jax.ShapeDtypeStruct((NC, np_rows, d), F32),
        mesh=mesh, scratch_types=scr,
        compiler_params=pltpu.CompilerParams(use_tc_tiling_on_sc=False))
    def run(data_hbm, idx_hbm, zero_hbm, out_hbm, *rest):
        slot_refs = [rest[4 * si:4 * (si + 1)] for si in range(slots)]
        acc = rest[-1]
        ci = lax.axis_index("c")
        si = lax.axis_index("s")
        wid = si * NC + ci
        row0 = pl.multiple_of(si * rps, 8)
        pltpu.sync_copy(zero_hbm, acc.at[pl.ds(row0, rps)])
        plsc.subcore_barrier()
        base = wid * b

        def do_chunks(j0, nact):
            descs = []
            for q in range(nact):
                iv, bv, s1, _ = slot_refs[q]
                off = pl.multiple_of(base + (j0 + q) * c, 8)
                pltpu.sync_copy(idx_hbm.at[pl.ds(off, c)], iv)
                descs.append(pltpu.async_copy(
                    data_hbm.at[pl.ds(off, c)], bv, s1))
            sc_d = []
            for q in range(nact):
                iv, bv, _, s2 = slot_refs[q]
                descs[q].wait()
                sc_d.append(pltpu.async_copy(bv, acc.at[iv], s2, add=True))
            for dsc in sc_d:
                dsc.wait()

        if slots == 2:
            lax.fori_loop(0, nch // 2,
                          lambda t, _: (do_chunks(t * 2, 2), 0)[1], 0)
            if nch % 2:
                do_chunks(nch - 1, 1)
        else:
            do_chunks(0, 1)
        plsc.subcore_barrier()
        pltpu.sync_copy(acc.at[pl.ds(row0, rps)],
                        out_hbm.at[ci, pl.ds(row0, rps)])

    return run(data, idx, zeros)


def _assign_from_bm(bm):
    na, nf = bm.shape
    blk = 256
    grid = (_cdiv(na, blk),)

    def body(bm_ref, out_ref):
        col = lax.broadcasted_iota(jnp.int32, (blk, nf), 1).astype(F32)
        out_ref[...] = jnp.sum(bm_ref[...] * col, axis=1,
                               keepdims=True).astype(jnp.int32)

    out = pl.pallas_call(
        body, grid=grid,
        in_specs=[pl.BlockSpec((blk, nf), lambda i: (i, 0))],
        out_specs=pl.BlockSpec((blk, 1), lambda i: (i, 0)),
        out_shape=jax.ShapeDtypeStruct((na, 1), jnp.int32))(bm)
    return out[:, 0]


def _proj2(h, wst, wtt, b0):
    n, hd = h.shape
    blk = 512
    grid = (_cdiv(n, blk),)

    def body(h_ref, ws_ref, wt_ref, b_ref, p_ref, q_ref):
        hh = h_ref[...]
        p_ref[...] = jnp.dot(hh, ws_ref[...],
                             preferred_element_type=F32) + b_ref[...]
        q_ref[...] = jnp.dot(hh, wt_ref[...], preferred_element_type=F32)

    full = pl.BlockSpec((hd, hd), lambda i: (0, 0))
    return pl.pallas_call(
        body, grid=grid,
        in_specs=[pl.BlockSpec((blk, hd), lambda i: (i, 0)), full, full,
                  pl.BlockSpec((1, hd), lambda i: (0, 0))],
        out_specs=[pl.BlockSpec((blk, hd), lambda i: (i, 0))] * 2,
        out_shape=[jax.ShapeDtypeStruct((n, hd), F32)] * 2)(h, wst, wtt, b0)


def _edge_mlp(g12, et2, attr2, dxy, tbl, wa, w1t, b1, watt, batt, n_real,
              trans=False):
    ep, hd = g12.shape
    blk = 1024
    grid = (ep // blk,)
    has_type = tbl is not None
    nt = tbl.shape[0] if has_type else 0
    d_out = 16 if trans else hd

    def body(*refs):
        if has_type:
            (g_ref, et_ref, a_ref, d_ref, tbl_ref, wa_ref, w1_ref, b1_ref,
             watt_ref, batt_ref, o_ref) = refs
        else:
            (g_ref, a_ref, d_ref, wa_ref, w1_ref, b1_ref, watt_ref,
             batt_ref, o_ref) = refs
        i = pl.program_id(0)
        dd = d_ref[...]
        radial = jnp.sum(dd * dd, axis=1, keepdims=True)
        pre = g12_val = g_ref[...]
        if has_type:
            oh = (et_ref[...] == lax.broadcasted_iota(
                jnp.int32, (blk, nt), 1)).astype(F32)
            pre = pre + jnp.dot(oh, tbl_ref[...], preferred_element_type=F32)
        pre = pre + radial * wa_ref[0:1, :] + a_ref[...] * wa_ref[1:2, :]
        t = jax.nn.silu(pre)
        mij = jax.nn.silu(jnp.dot(t, w1_ref[...],
                                  preferred_element_type=F32) + b1_ref[...])
        rowid = i * blk + lax.broadcasted_iota(jnp.int32, (blk, 1), 0)
        mask = rowid < n_real
        if trans:
            s = jnp.dot(mij, watt_ref[...], preferred_element_type=F32)
            s = s + batt_ref[...]
            cd = dd / (jnp.sqrt(radial + 1e-8) + 1.0)
            out = cd * (jnp.tanh(s) * COORDS_RANGE)
        else:
            att = jax.nn.sigmoid(
                jnp.dot(mij, watt_ref[...], preferred_element_type=F32)
                + batt_ref[...])
            out = mij * att
        o_ref[...] = jnp.where(mask, out, 0.0)

    eblk = pl.BlockSpec((blk, hd), lambda i: (i, 0))
    one = pl.BlockSpec((blk, 1), lambda i: (i, 0))
    full = pl.BlockSpec((hd, hd), lambda i: (0, 0))
    in_specs = [eblk]
    args = [g12]
    if has_type:
        in_specs.append(one)
        args.append(et2)
    in_specs += [one, pl.BlockSpec((blk, 16), lambda i: (i, 0))]
    args += [attr2, dxy]
    if has_type:
        in_specs.append(pl.BlockSpec((nt, hd), lambda i: (0, 0)))
        args.append(tbl)
    in_specs += [pl.BlockSpec((2, hd), lambda i: (0, 0)), full,
                 pl.BlockSpec((1, hd), lambda i: (0, 0)),
                 pl.BlockSpec((hd, 1), lambda i: (0, 0)),
                 pl.BlockSpec((1, 1), lambda i: (0, 0))]
    args += [wa, w1t, b1, watt, batt]
    return pl.pallas_call(
        body, grid=grid, in_specs=in_specs,
        out_specs=pl.BlockSpec((blk, d_out), lambda i: (i, 0)),
        out_shape=jax.ShapeDtypeStruct((ep, d_out), F32))(*args)


def _node_mlp(h, aux, agg0, agg1, n0at, n0bt, n0ct, b0, n1t, b1):
    n, hd = h.shape
    blk = 512
    grid = (_cdiv(n, blk),)
    n_aux = len(aux)

    def body(*refs):
        h_ref = refs[0]
        aux_refs = refs[1:1 + n_aux]
        a0_ref, a1_ref, wa_ref, wb_ref, wc_ref, b0_ref, w1_ref, b1_ref, \
            o_ref = refs[1 + n_aux:]
        hh = h_ref[...]
        av = aux_refs[0][...]
        if n_aux == 2:
            av = av + aux_refs[1][...]
        agg = (a0_ref[...] + a1_ref[...]) * INV_NORM
        pre = (jnp.dot(hh, wa_ref[...], preferred_element_type=F32)
               + jnp.dot(av, wb_ref[...], preferred_element_type=F32)
               + jnp.dot(agg, wc_ref[...], preferred_element_type=F32)
               + b0_ref[...])
        t = jax.nn.silu(pre)
        o_ref[...] = hh + jnp.dot(t, w1_ref[...],
                                  preferred_element_type=F32) + b1_ref[...]

    rblk = pl.BlockSpec((blk, hd), lambda i: (i, 0))
    full = pl.BlockSpec((hd, hd), lambda i: (0, 0))
    bias = pl.BlockSpec((1, hd), lambda i: (0, 0))
    in_specs = [rblk] * (1 + n_aux) + [rblk, rblk, full, full, full, bias,
                                       full, bias]
    return pl.pallas_call(
        body, grid=grid, in_specs=in_specs, out_specs=rblk,
        out_shape=jax.ShapeDtypeStruct((n, hd), F32))(
            h, *aux, agg0, agg1, n0at, n0bt, n0ct, b0, n1t, b1)


def _final_coords(xa16, h_a, hfa, xfa16, at0, at1, f0at, f0bt, wfr, b0, f1t,
                  b1, f2w):
    n, hd = h_a.shape
    blk = 512
    grid = (_cdiv(n, blk),)

    def body(x_ref, h_ref, hf_ref, xf_ref, a0_ref, a1_ref, wa_ref, wb_ref,
             wr_ref, b0_ref, w1_ref, b1_ref, w2_ref, o_ref):
        x = x_ref[...]
        cdf = x - xf_ref[...]
        radial = jnp.sum(cdf * cdf, axis=1, keepdims=True)
        cdfn = cdf / (jnp.sqrt(radial + 1e-8) + 1.0)
        pre = (jnp.dot(h_ref[...], wa_ref[...], preferred_element_type=F32)
               + jnp.dot(hf_ref[...], wb_ref[...], preferred_element_type=F32)
               + radial * wr_ref[...] + b0_ref[...])
        t = jax.nn.silu(pre)
        u = jax.nn.silu(jnp.dot(t, w1_ref[...],
                                preferred_element_type=F32) + b1_ref[...])
        s = jnp.dot(u, w2_ref[...], preferred_element_type=F32)
        trans = cdfn * (jnp.tanh(s) * COORDS_RANGE)
        agg = (a0_ref[...] + a1_ref[...]) * INV_NORM
        o_ref[...] = x + agg + trans

    c16 = pl.BlockSpec((blk, 16), lambda i: (i, 0))
    rblk = pl.BlockSpec((blk, hd), lambda i: (i, 0))
    full = pl.BlockSpec((hd, hd), lambda i: (0, 0))
    bias = pl.BlockSpec((1, hd), lambda i: (0, 0))
    return pl.pallas_call(
        body, grid=grid,
        in_specs=[c16, rblk, rblk, c16, c16, c16, full, full, bias, bias,
                  full, bias, pl.BlockSpec((hd, 1), lambda i: (0, 0))],
        out_specs=c16,
        out_shape=jax.ShapeDtypeStruct((n, 16), F32))(
            xa16, h_a, hfa, xfa16, at0, at1, f0at, f0bt, wfr, b0, f1t, b1,
            f2w)


def _gcl_weights(p, hd, with_type):
    w0 = p['edge_mlp0']['w']
    ws = w0[:, :hd].T
    wt = w0[:, hd:2 * hd].T
    wa = w0[:, 2 * hd:2 * hd + 2].T
    b0 = p['edge_mlp0']['b'][None, :]
    tbl = None
    if with_type:
        wb = w0[:, 2 * hd + 2:]
        tbl = jnp.dot(p['bond_emb'], wb.T)
    w1t = p['edge_mlp1']['w'].T
    b1 = p['edge_mlp1']['b'][None, :]
    watt = p['att_mlp']['w'].T
    batt = p['att_mlp']['b'].reshape(1, 1)
    n0 = p['node_mlp0']['w']
    n0at = n0[:, :hd].T
    n0bt = n0[:, hd:2 * hd].T
    n0ct = n0[:, 2 * hd:].T
    nb0 = p['node_mlp0']['b'][None, :]
    n1t = p['node_mlp1']['w'].T
    nb1 = p['node_mlp1']['b'][None, :]
    return dict(ws=ws, wt=wt, wa=wa, b0=b0, tbl=tbl, w1t=w1t, b1=b1,
                watt=watt, batt=batt, n0at=n0at, n0bt=n0bt, n0ct=n0ct,
                nb0=nb0, n1t=n1t, nb1=nb1)


def kernel(h_a, x_a, e_a_idx, e_a_type, e_a_attr, h_f, x_f, e_f_idx,
           e_f_attr, m_mat, bm_mat, params):
    na, hd = h_a.shape
    nf = h_f.shape[0]
    ea = e_a_idx.shape[1]
    ef = e_f_idx.shape[1]
    nl = len(params['f_inv'])

    eap, _ = _pad_len(ea)
    efp, _ = _pad_len(ef)
    nap, _ = _pad_len(na)

    def pad1(v, n):
        return _pad_rows(v.astype(jnp.int32), n)

    row_a = pad1(e_a_idx[0], eap)
    col_a = pad1(e_a_idx[1], eap)
    row_f = pad1(e_f_idx[0], efp)
    col_f = pad1(e_f_idx[1], efp)
    et2 = _pad_rows(e_a_type.astype(jnp.int32)[:, None], eap)
    attr_a2 = _pad_rows(e_a_attr, eap)
    attr_f2 = _pad_rows(e_f_attr, efp)

    assign = _assign_from_bm(bm_mat)
    assign_p = pad1(assign, nap)

    xa16 = jnp.pad(x_a, ((0, 0), (0, 13)))
    xf16 = jnp.pad(x_f, ((0, 0), (0, 13)))

    dxy_a, dxy_f, xfa16 = _sc_gathers([
        ((xa16, -xa16), (row_a, col_a)),
        ((xf16, -xf16), (row_f, col_f)),
        ((xf16,), (assign_p,)),
    ])

    for i in range(nl):
        wf = _gcl_weights(params['f_inv'][i], hd, with_type=False)
        wa_ = _gcl_weights(params['a_inv'][i], hd, with_type=True)

        hs = _sc_scatter_add(_pad_rows(h_a, nap), assign_p, nf)

        pf, qf = _proj2(h_f, wf['ws'], wf['wt'], wf['b0'])
        (gf12,) = _sc_gathers([((pf, qf), (row_f, col_f))])
        eff = _edge_mlp(gf12, None, attr_f2, dxy_f, None, wf['wa'],
                        wf['w1t'], wf['b1'], wf['watt'], wf['batt'], ef)
        af = _sc_scatter_add(eff, row_f, nf)
        h_f = _node_mlp(h_f, (hs[0], hs[1]), af[0], af[1], wf['n0at'],
                        wf['n0bt'], wf['n0ct'], wf['nb0'], wf['n1t'],
                        wf['nb1'])

        pa, qa = _proj2(h_a, wa_['ws'], wa_['wt'], wa_['b0'])
        ga12, hfa = _sc_gathers([((pa, qa), (row_a, col_a)),
                                 ((h_f,), (assign_p,))])
        eaf = _edge_mlp(ga12, et2, attr_a2, dxy_a, wa_['tbl'], wa_['wa'],
                        wa_['w1t'], wa_['b1'], wa_['watt'], wa_['batt'], ea)
        aa = _sc_scatter_add(eaf, row_a, na)
        h_a = _node_mlp(h_a, (hfa,), aa[0], aa[1], wa_['n0at'], wa_['n0bt'],
                        wa_['n0ct'], wa_['nb0'], wa_['n1t'], wa_['nb1'])

    pe = params['a_eq']
    w0 = pe['atom0']['w']
    wse = w0[:, :hd].T
    wte = w0[:, hd:2 * hd].T
    wae = w0[:, 2 * hd:2 * hd + 2].T
    tbl_e = jnp.dot(pe['bond_emb'], w0[:, 2 * hd + 2:].T)
    b0e = pe['atom0']['b'][None, :]
    a1t = pe['atom1']['w'].T
    ab1 = pe['atom1']['b'][None, :]
    a2w = pe['atom2_w'].T
    zb = jnp.zeros((1, 1), F32)

    p_e, q_e = _proj2(h_a, wse, wte, b0e)
    ge12, hfa_fin = _sc_gathers([((p_e, q_e), (row_a, col_a)),
                                 ((h_f,), (assign_p,))])
    tr = _edge_mlp(ge12, et2, attr_a2, dxy_a, tbl_e, wae, a1t, ab1, a2w, zb,
                   ea, trans=True)
    at = _sc_scatter_add(tr, row_a, na)

    f0 = pe['frag0']['w']
    f0at = f0[:, :hd].T
    f0bt = f0[:, hd:2 * hd].T
    wfr = (f0[:, 2 * hd] + f0[:, 2 * hd + 1])[None, :]
    fb0 = pe['frag0']['b'][None, :]
    f1t = pe['frag1']['w'].T
    fb1 = pe['frag1']['b'][None, :]
    f2w = pe['frag2_w'].T

    xout16 = _final_coords(xa16, h_a, hfa_fin, xfa16, at[0], at[1], f0at,
                           f0bt, wfr, fb0, f1t, fb1, f2w)
    return h_a, xout16[:, :3]


# speedup vs baseline: 2.1209x; 1.0403x over previous
import functools

import jax
import jax.numpy as jnp
from jax import lax
from jax.experimental import pallas as pl
from jax.experimental.pallas import tpu as pltpu
from jax.experimental.pallas import tpu_sc as plsc

F32 = jnp.float32
NC, NS = 2, 16
NW = NC * NS
COORDS_RANGE = 15.0
INV_NORM = 0.01


def _cdiv(a, b):
    return (a + b - 1) // b


def _pad_rows(x, n):
    if x.shape[0] == n:
        return x
    pad = [(0, n - x.shape[0])] + [(0, 0)] * (x.ndim - 1)
    return jnp.pad(x, pad)


def _pad_len(e):
    for c in (128, 64, 32, 16, 8):
        ep = _cdiv(e, NW * c) * NW * c
        if ep - e <= max(e // 16, NW * 8):
            return ep, c
    return _cdiv(e, NW * 8) * NW * 8, 8


def _chunk_of(e):
    b = e // NW
    return max(c for c in (8, 16, 32, 64, 128) if b % c == 0)


_SPMEM_BUDGET = 6_600_000


def _pick_chunk(b, d, budget, slots):
    cmax = budget // (NS * slots * (d * 4 + 4))
    best = 8
    for c in range(8, min(128, b) + 1, 8):
        if b % c == 0 and c <= cmax:
            best = c
    return best


def _sc_gathers(specs):
    plans = []
    flat_in = []
    out_type = []
    scratch = []
    budget = _SPMEM_BUDGET
    for tables, idxs in specs:
        e = idxs[0].shape[0]
        d = tables[0].shape[1]
        b = e // NW
        c = _pick_chunk(b, d, budget, slots=4)
        nch = b // c
        slots = min(4, nch)
        budget -= NS * slots * c * (d * 4 + 4)
        plans.append((e, d, b, c, nch, slots, len(tables)))
        flat_in += list(tables) + list(idxs)
        out_type.append(jax.ShapeDtypeStruct((e, d), F32))
        group = []
        for _ in range(slots):
            group += [pltpu.VMEM((c,), jnp.int32) for _ in tables]
            group += [pltpu.VMEM((c, d), F32), pltpu.SemaphoreType.DMA,
                      pltpu.SemaphoreType.DMA]
        scratch.append(group)
    n_in = len(flat_in)
    mesh = plsc.VectorSubcoreMesh(core_axis_name="c", subcore_axis_name="s")
    flat_scratch = [s for group in scratch for s in group]

    @functools.partial(
        pl.kernel, out_type=tuple(out_type), mesh=mesh,
        scratch_types=flat_scratch,
        compiler_params=pltpu.CompilerParams(use_tc_tiling_on_sc=False))
    def run(*refs):
        wid = lax.axis_index("s") * NC + lax.axis_index("c")
        ipos = 0
        spos = n_in + len(plans)
        for kk, (e, d, b, c, nch, slots, ntab) in enumerate(plans):
            t_refs = refs[ipos:ipos + ntab]
            i_refs = refs[ipos + ntab:ipos + 2 * ntab]
            o_ref = refs[n_in + kk]
            per = ntab + 3
            slot_refs = [refs[spos + si * per:spos + (si + 1) * per]
                         for si in range(slots)]
            ipos += 2 * ntab
            spos += slots * per
            base = wid * b

            def do_wave(j0, nact, t_refs=t_refs, i_refs=i_refs,
                        o_ref=o_ref, slot_refs=slot_refs, base=base, c=c,
                        ntab=ntab):
                descs = []
                for si in range(nact):
                    sr = slot_refs[si]
                    off = pl.multiple_of(base + (j0 + si) * c, 8)
                    for q in range(ntab):
                        pltpu.sync_copy(i_refs[q].at[pl.ds(off, c)], sr[q])
                    descs.append(pltpu.async_copy(t_refs[0].at[sr[0]],
                                                  sr[ntab], sr[ntab + 1]))
                if ntab == 2:
                    add_d = []
                    for si in range(nact):
                        sr = slot_refs[si]
                        descs[si].wait()
                        add_d.append(pltpu.async_copy(
                            t_refs[1].at[sr[1]], sr[ntab], sr[ntab + 1],
                            add=True))
                    descs = add_d
                wb = []
                for si in range(nact):
                    sr = slot_refs[si]
                    off = pl.multiple_of(base + (j0 + si) * c, 8)
                    descs[si].wait()
                    wb.append(pltpu.async_copy(
                        sr[ntab], o_ref.at[pl.ds(off, c)], sr[ntab + 2]))
                for dsc in wb:
                    dsc.wait()

            if nch // slots:
                lax.fori_loop(0, nch // slots,
                              lambda t, _, s=slots: (do_wave(t * s, s),
                                                     0)[1], 0)
            if nch % slots:
                do_wave(nch - nch % slots, nch % slots)

    outs = run(*flat_in)
    return outs if isinstance(outs, (tuple, list)) else (outs,)


def _sc_scatter_add(data, idx, n_rows):
    e, d = data.shape
    b = e // NW
    np_rows = _cdiv(n_rows, NS * 8) * NS * 8
    rps = np_rows // NS
    c = _pick_chunk(b, d, _SPMEM_BUDGET - np_rows * d * 4, slots=2)
    nch = b // c
    slots = 2 if nch >= 2 else 1
    zeros = jnp.zeros((rps, d), F32)
    mesh = plsc.VectorSubcoreMesh(core_axis_name="c", subcore_axis_name="s")

    scr = []
    for _ in range(slots):
        scr += [pltpu.VMEM((c,), jnp.int32), pltpu.VMEM((c, d), F32),
                pltpu.SemaphoreType.DMA, pltpu.SemaphoreType.DMA]
    scr.append(pltpu.VMEM_SHARED((np_rows, d), F32))

    @functools.partial(
        pl.kernel,
        out_type=jax.ShapeDtypeStruct((NC, np_rows, d), F32),
        mesh=mesh, scratch_types=scr,
        compiler_params=pltpu.CompilerParams(use_tc_tiling_on_sc=False))
    def run(data_hbm, idx_hbm, zero_hbm, out_hbm, *rest):
        slot_refs = [rest[4 * si:4 * (si + 1)] for si in range(slots)]
        acc = rest[-1]
        ci = lax.axis_index("c")
        si = lax.axis_index("s")
        wid = si * NC + ci
        row0 = pl.multiple_of(si * rps, 8)
        pltpu.sync_copy(zero_hbm, acc.at[pl.ds(row0, rps)])
        plsc.subcore_barrier()
        base = wid * b

        def do_chunks(j0, nact):
            descs = []
            for q in range(nact):
                iv, bv, s1, _ = slot_refs[q]
                off = pl.multiple_of(base + (j0 + q) * c, 8)
                pltpu.sync_copy(idx_hbm.at[pl.ds(off, c)], iv)
                descs.append(pltpu.async_copy(
                    data_hbm.at[pl.ds(off, c)], bv, s1))
            sc_d = []
            for q in range(nact):
                iv, bv, _, s2 = slot_refs[q]
                descs[q].wait()
                sc_d.append(pltpu.async_copy(bv, acc.at[iv], s2, add=True))
            for dsc in sc_d:
                dsc.wait()

        if slots == 2:
            lax.fori_loop(0, nch // 2,
                          lambda t, _: (do_chunks(t * 2, 2), 0)[1], 0)
            if nch % 2:
                do_chunks(nch - 1, 1)
        else:
            do_chunks(0, 1)
        plsc.subcore_barrier()
        pltpu.sync_copy(acc.at[pl.ds(row0, rps)],
                        out_hbm.at[ci, pl.ds(row0, rps)])

    return run(data, idx, zeros)


def _assign_from_bm(bm):
    na, nf = bm.shape
    blk = 256
    grid = (_cdiv(na, blk),)

    def body(bm_ref, out_ref):
        col = lax.broadcasted_iota(jnp.int32, (blk, nf), 1).astype(F32)
        out_ref[...] = jnp.sum(bm_ref[...] * col, axis=1,
                               keepdims=True).astype(jnp.int32)

    out = pl.pallas_call(
        body, grid=grid,
        in_specs=[pl.BlockSpec((blk, nf), lambda i: (i, 0))],
        out_specs=pl.BlockSpec((blk, 1), lambda i: (i, 0)),
        out_shape=jax.ShapeDtypeStruct((na, 1), jnp.int32))(bm)
    return out[:, 0]


def _proj2(h, wst, wtt, b0):
    n, hd = h.shape
    blk = 512
    grid = (_cdiv(n, blk),)

    def body(h_ref, ws_ref, wt_ref, b_ref, p_ref, q_ref):
        hh = h_ref[...]
        p_ref[...] = jnp.dot(hh, ws_ref[...],
                             preferred_element_type=F32) + b_ref[...]
        q_ref[...] = jnp.dot(hh, wt_ref[...], preferred_element_type=F32)

    full = pl.BlockSpec((hd, hd), lambda i: (0, 0))
    return pl.pallas_call(
        body, grid=grid,
        in_specs=[pl.BlockSpec((blk, hd), lambda i: (i, 0)), full, full,
                  pl.BlockSpec((1, hd), lambda i: (0, 0))],
        out_specs=[pl.BlockSpec((blk, hd), lambda i: (i, 0))] * 2,
        out_shape=[jax.ShapeDtypeStruct((n, hd), F32)] * 2)(h, wst, wtt, b0)


def _edge_mlp(g12, et2, attr2, dxy, tbl, wa, w1t, b1, watt, batt, n_real,
              trans=False):
    ep, hd = g12.shape
    blk = 1024
    grid = (ep // blk,)
    has_type = tbl is not None
    nt = tbl.shape[0] if has_type else 0
    d_out = 16 if trans else hd

    def body(*refs):
        if has_type:
            (g_ref, et_ref, a_ref, d_ref, tbl_ref, wa_ref, w1_ref, b1_ref,
             watt_ref, batt_ref, o_ref) = refs
        else:
            (g_ref, a_ref, d_ref, wa_ref, w1_ref, b1_ref, watt_ref,
             batt_ref, o_ref) = refs
        i = pl.program_id(0)
        dd = d_ref[...]
        radial = jnp.sum(dd * dd, axis=1, keepdims=True)
        pre = g12_val = g_ref[...]
        if has_type:
            oh = (et_ref[...] == lax.broadcasted_iota(
                jnp.int32, (blk, nt), 1)).astype(F32)
            pre = pre + jnp.dot(oh, tbl_ref[...], preferred_element_type=F32)
        pre = pre + radial * wa_ref[0:1, :] + a_ref[...] * wa_ref[1:2, :]
        t = jax.nn.silu(pre)
        mij = jax.nn.silu(jnp.dot(t, w1_ref[...],
                                  preferred_element_type=F32) + b1_ref[...])
        rowid = i * blk + lax.broadcasted_iota(jnp.int32, (blk, 1), 0)
        mask = rowid < n_real
        if trans:
            s = jnp.dot(mij, watt_ref[...], preferred_element_type=F32)
            s = s + batt_ref[...]
            cd = dd / (jnp.sqrt(radial + 1e-8) + 1.0)
            out = cd * (jnp.tanh(s) * COORDS_RANGE)
        else:
            att = jax.nn.sigmoid(
                jnp.dot(mij, watt_ref[...], preferred_element_type=F32)
                + batt_ref[...])
            out = mij * att
        o_ref[...] = jnp.where(mask, out, 0.0)

    eblk = pl.BlockSpec((blk, hd), lambda i: (i, 0))
    one = pl.BlockSpec((blk, 1), lambda i: (i, 0))
    full = pl.BlockSpec((hd, hd), lambda i: (0, 0))
    in_specs = [eblk]
    args = [g12]
    if has_type:
        in_specs.append(one)
        args.append(et2)
    in_specs += [one, pl.BlockSpec((blk, 16), lambda i: (i, 0))]
    args += [attr2, dxy]
    if has_type:
        in_specs.append(pl.BlockSpec((nt, hd), lambda i: (0, 0)))
        args.append(tbl)
    in_specs += [pl.BlockSpec((2, hd), lambda i: (0, 0)), full,
                 pl.BlockSpec((1, hd), lambda i: (0, 0)),
                 pl.BlockSpec((hd, 1), lambda i: (0, 0)),
                 pl.BlockSpec((1, 1), lambda i: (0, 0))]
    args += [wa, w1t, b1, watt, batt]
    return pl.pallas_call(
        body, grid=grid, in_specs=in_specs,
        out_specs=pl.BlockSpec((blk, d_out), lambda i: (i, 0)),
        out_shape=jax.ShapeDtypeStruct((ep, d_out), F32))(*args)


def _node_mlp(h, aux, agg0, agg1, n0at, n0bt, n0ct, b0, n1t, b1):
    n, hd = h.shape
    blk = 512
    grid = (_cdiv(n, blk),)
    n_aux = len(aux)

    def body(*refs):
        h_ref = refs[0]
        aux_refs = refs[1:1 + n_aux]
        a0_ref, a1_ref, wa_ref, wb_ref, wc_ref, b0_ref, w1_ref, b1_ref, \
            o_ref = refs[1 + n_aux:]
        hh = h_ref[...]
        av = aux_refs[0][...]
        if n_aux == 2:
            av = av + aux_refs[1][...]
        agg = (a0_ref[...] + a1_ref[...]) * INV_NORM
        pre = (jnp.dot(hh, wa_ref[...], preferred_element_type=F32)
               + jnp.dot(av, wb_ref[...], preferred_element_type=F32)
               + jnp.dot(agg, wc_ref[...], preferred_element_type=F32)
               + b0_ref[...])
        t = jax.nn.silu(pre)
        o_ref[...] = hh + jnp.dot(t, w1_ref[...],
                                  preferred_element_type=F32) + b1_ref[...]

    rblk = pl.BlockSpec((blk, hd), lambda i: (i, 0))
    full = pl.BlockSpec((hd, hd), lambda i: (0, 0))
    bias = pl.BlockSpec((1, hd), lambda i: (0, 0))
    in_specs = [rblk] * (1 + n_aux) + [rblk, rblk, full, full, full, bias,
                                       full, bias]
    return pl.pallas_call(
        body, grid=grid, in_specs=in_specs, out_specs=rblk,
        out_shape=jax.ShapeDtypeStruct((n, hd), F32))(
            h, *aux, agg0, agg1, n0at, n0bt, n0ct, b0, n1t, b1)


def _final_coords(xa16, h_a, hfa, xfa16, at0, at1, f0at, f0bt, wfr, b0, f1t,
                  b1, f2w):
    n, hd = h_a.shape
    blk = 512
    grid = (_cdiv(n, blk),)

    def body(x_ref, h_ref, hf_ref, xf_ref, a0_ref, a1_ref, wa_ref, wb_ref,
             wr_ref, b0_ref, w1_ref, b1_ref, w2_ref, o_ref):
        x = x_ref[...]
        cdf = x - xf_ref[...]
        radial = jnp.sum(cdf * cdf, axis=1, keepdims=True)
        cdfn = cdf / (jnp.sqrt(radial + 1e-8) + 1.0)
        pre = (jnp.dot(h_ref[...], wa_ref[...], preferred_element_type=F32)
               + jnp.dot(hf_ref[...], wb_ref[...], preferred_element_type=F32)
               + radial * wr_ref[...] + b0_ref[...])
        t = jax.nn.silu(pre)
        u = jax.nn.silu(jnp.dot(t, w1_ref[...],
                                preferred_element_type=F32) + b1_ref[...])
        s = jnp.dot(u, w2_ref[...], preferred_element_type=F32)
        trans = cdfn * (jnp.tanh(s) * COORDS_RANGE)
        agg = (a0_ref[...] + a1_ref[...]) * INV_NORM
        o_ref[...] = x + agg + trans

    c16 = pl.BlockSpec((blk, 16), lambda i: (i, 0))
    rblk = pl.BlockSpec((blk, hd), lambda i: (i, 0))
    full = pl.BlockSpec((hd, hd), lambda i: (0, 0))
    bias = pl.BlockSpec((1, hd), lambda i: (0, 0))
    return pl.pallas_call(
        body, grid=grid,
        in_specs=[c16, rblk, rblk, c16, c16, c16, full, full, bias, bias,
                  full, bias, pl.BlockSpec((hd, 1), lambda i: (0, 0))],
        out_specs=c16,
        out_shape=jax.ShapeDtypeStruct((n, 16), F32))(
            xa16, h_a, hfa, xfa16, at0, at1, f0at, f0bt, wfr, b0, f1t, b1,
            f2w)


def _gcl_weights(p, hd, with_type):
    w0 = p['edge_mlp0']['w']
    ws = w0[:, :hd].T
    wt = w0[:, hd:2 * hd].T
    wa = w0[:, 2 * hd:2 * hd + 2].T
    b0 = p['edge_mlp0']['b'][None, :]
    tbl = None
    if with_type:
        wb = w0[:, 2 * hd + 2:]
        tbl = jnp.dot(p['bond_emb'], wb.T)
    w1t = p['edge_mlp1']['w'].T
    b1 = p['edge_mlp1']['b'][None, :]
    watt = p['att_mlp']['w'].T
    batt = p['att_mlp']['b'].reshape(1, 1)
    n0 = p['node_mlp0']['w']
    n0at = n0[:, :hd].T
    n0bt = n0[:, hd:2 * hd].T
    n0ct = n0[:, 2 * hd:].T
    nb0 = p['node_mlp0']['b'][None, :]
    n1t = p['node_mlp1']['w'].T
    nb1 = p['node_mlp1']['b'][None, :]
    return dict(ws=ws, wt=wt, wa=wa, b0=b0, tbl=tbl, w1t=w1t, b1=b1,
                watt=watt, batt=batt, n0at=n0at, n0bt=n0bt, n0ct=n0ct,
                nb0=nb0, n1t=n1t, nb1=nb1)


def kernel(h_a, x_a, e_a_idx, e_a_type, e_a_attr, h_f, x_f, e_f_idx,
           e_f_attr, m_mat, bm_mat, params):
    na, hd = h_a.shape
    nf = h_f.shape[0]
    ea = e_a_idx.shape[1]
    ef = e_f_idx.shape[1]
    nl = len(params['f_inv'])

    eap, _ = _pad_len(ea)
    efp, _ = _pad_len(ef)
    nap, _ = _pad_len(na)

    def pad1(v, n):
        return _pad_rows(v.astype(jnp.int32), n)

    row_a = pad1(e_a_idx[0], eap)
    col_a = pad1(e_a_idx[1], eap)
    row_f = pad1(e_f_idx[0], efp)
    col_f = pad1(e_f_idx[1], efp)
    et2 = _pad_rows(e_a_type.astype(jnp.int32)[:, None], eap)
    attr_a2 = _pad_rows(e_a_attr, eap)
    attr_f2 = _pad_rows(e_f_attr, efp)

    assign = _assign_from_bm(bm_mat)
    assign_p = pad1(assign, nap)

    xa16 = jnp.pad(x_a, ((0, 0), (0, 13)))
    xf16 = jnp.pad(x_f, ((0, 0), (0, 13)))

    dxy_a, dxy_f, xfa16 = _sc_gathers([
        ((xa16, -xa16), (row_a, col_a)),
        ((xf16, -xf16), (row_f, col_f)),
        ((xf16,), (assign_p,)),
    ])

    for i in range(nl):
        wf = _gcl_weights(params['f_inv'][i], hd, with_type=False)
        wa_ = _gcl_weights(params['a_inv'][i], hd, with_type=True)

        hs = _sc_scatter_add(_pad_rows(h_a, nap), assign_p, nf)

        pf, qf = _proj2(h_f, wf['ws'], wf['wt'], wf['b0'])
        (gf12,) = _sc_gathers([((pf, qf), (row_f, col_f))])
        eff = _edge_mlp(gf12, None, attr_f2, dxy_f, None, wf['wa'],
                        wf['w1t'], wf['b1'], wf['watt'], wf['batt'], ef)
        af = _sc_scatter_add(eff, row_f, nf)
        h_f = _node_mlp(h_f, (hs[0], hs[1]), af[0], af[1], wf['n0at'],
                        wf['n0bt'], wf['n0ct'], wf['nb0'], wf['n1t'],
                        wf['nb1'])

        pa, qa = _proj2(h_a, wa_['ws'], wa_['wt'], wa_['b0'])
        ga12, hfa = _sc_gathers([((pa, qa), (row_a, col_a)),
                                 ((h_f,), (assign_p,))])
        eaf = _edge_mlp(ga12, et2, attr_a2, dxy_a, wa_['tbl'], wa_['wa'],
                        wa_['w1t'], wa_['b1'], wa_['watt'], wa_['batt'], ea)
        aa = _sc_scatter_add(eaf, row_a, na)
        h_a = _node_mlp(h_a, (hfa,), aa[0], aa[1], wa_['n0at'], wa_['n0bt'],
                        wa_['n0ct'], wa_['nb0'], wa_['n1t'], wa_['nb1'])

    pe = params['a_eq']
    w0 = pe['atom0']['w']
    wse = w0[:, :hd].T
    wte = w0[:, hd:2 * hd].T
    wae = w0[:, 2 * hd:2 * hd + 2].T
    tbl_e = jnp.dot(pe['bond_emb'], w0[:, 2 * hd + 2:].T)
    b0e = pe['atom0']['b'][None, :]
    a1t = pe['atom1']['w'].T
    ab1 = pe['atom1']['b'][None, :]
    a2w = pe['atom2_w'].T
    zb = jnp.zeros((1, 1), F32)

    p_e, q_e = _proj2(h_a, wse, wte, b0e)
    ge12, hfa_fin = _sc_gathers([((p_e, q_e), (row_a, col_a)),
                                 ((h_f,), (assign_p,))])
    tr = _edge_mlp(ge12, et2, attr_a2, dxy_a, tbl_e, wae, a1t, ab1, a2w, zb,
                   ea, trans=True)
    at = _sc_scatter_add(tr, row_a, na)

    f0 = pe['frag0']['w']
    f0at = f0[:, :hd].T
    f0bt = f0[:, hd:2 * hd].T
    wfr = (f0[:, 2 * hd] + f0[:, 2 * hd + 1])[None, :]
    fb0 = pe['frag0']['b'][None, :]
    f1t = pe['frag1']['w'].T
    fb1 = pe['frag1']['b'][None, :]
    f2w = pe['frag2_w'].T

    xout16 = _final_coords(xa16, h_a, hfa_fin, xfa16, at[0], at[1], f0at,
                           f0bt, wfr, fb0, f1t, fb1, f2w)
    return h_a, xout16[:, :3]
